# Initial kernel scaffold; baseline (speedup 1.0000x reference)
import jax
import jax.numpy as jnp
from jax.experimental import pallas as pl


def kernel(seq_len, weight):
    raise NotImplementedError("write your pallas kernel here")


# speedup vs baseline: 2.2982x; 2.2982x over previous
import jax
import jax.numpy as jnp
from jax.experimental import pallas as pl
from jax.experimental.pallas import tpu as pltpu

_BLK = 256


def _embed_kernel(s_ref, w_ref, c_ref, o_ref):
    i = pl.program_id(0)
    s = s_ref[0]
    blk = o_ref.shape[0]
    end = (i + 1) * blk

    @pl.when(end <= s)
    def _copy():
        o_ref[...] = w_ref[...]

    @pl.when(end > s)
    def _clamp():
        rows = i * blk + jax.lax.broadcasted_iota(jnp.int32, o_ref.shape, 0)
        clamp_row = c_ref[pl.ds((s - 1) % blk, 1), :]
        o_ref[...] = jnp.where(rows < s, w_ref[...], clamp_row)


def kernel(seq_len, weight):
    n, d = weight.shape
    blk = min(_BLK, n)
    s = jnp.asarray(seq_len, jnp.int32).reshape(1)
    return pl.pallas_call(
        _embed_kernel,
        grid_spec=pltpu.PrefetchScalarGridSpec(
            num_scalar_prefetch=1,
            grid=(n // blk,),
            in_specs=[
                pl.BlockSpec((blk, d), lambda i, sp: (i, 0)),
                pl.BlockSpec((blk, d), lambda i, sp: ((sp[0] - 1) // blk, 0)),
            ],
            out_specs=pl.BlockSpec((blk, d), lambda i, sp: (i, 0)),
        ),
        out_shape=jax.ShapeDtypeStruct((n, d), weight.dtype),
        compiler_params=pltpu.CompilerParams(
            dimension_semantics=("arbitrary",),
        ),
    )(s, weight, weight)


# speedup vs baseline: 2.3110x; 1.0056x over previous
import jax
import jax.numpy as jnp
from jax.experimental import pallas as pl
from jax.experimental.pallas import tpu as pltpu

_BLK = 512


def _embed_kernel(s_ref, w_ref, c_ref, o_ref):
    i = pl.program_id(0)
    s = s_ref[0]
    blk = o_ref.shape[0]
    end = (i + 1) * blk

    @pl.when(end <= s)
    def _copy():
        o_ref[...] = w_ref[...]

    @pl.when(end > s)
    def _clamp():
        rows = i * blk + jax.lax.broadcasted_iota(jnp.int32, o_ref.shape, 0)
        clamp_row = c_ref[pl.ds((s - 1) % blk, 1), :]
        o_ref[...] = jnp.where(rows < s, w_ref[...], clamp_row)


def kernel(seq_len, weight):
    n, d = weight.shape
    blk = min(_BLK, n)
    s = jnp.asarray(seq_len, jnp.int32).reshape(1)
    return pl.pallas_call(
        _embed_kernel,
        grid_spec=pltpu.PrefetchScalarGridSpec(
            num_scalar_prefetch=1,
            grid=(n // blk,),
            in_specs=[
                pl.BlockSpec((blk, d), lambda i, sp: (i, 0)),
                pl.BlockSpec((blk, d), lambda i, sp: ((sp[0] - 1) // blk, 0)),
            ],
            out_specs=pl.BlockSpec((blk, d), lambda i, sp: (i, 0)),
        ),
        out_shape=jax.ShapeDtypeStruct((n, d), weight.dtype),
        compiler_params=pltpu.CompilerParams(
            dimension_semantics=("parallel",),
        ),
    )(s, weight, weight)
